# baseline (device time: 460028 ns/iter reference)
import jax
import jax.numpy as jnp
from jax import lax
from jax.experimental import pallas as pl
from jax.experimental.pallas import tpu as pltpu

M, N = 32768, 1024
HALF = M // 2
CH = 2048
NC = HALF // CH

MESH = pl.DeviceIdType.MESH


def kernel(x):
    def body(x_ref, out_ref, loc, ysend, yrecv, xsend, xrecv,
             load_sems, smine_sems, sother_sems,
             ysend_sems, yrecv_sems, xsend_sems, xrecv_sems,
             y_credit, x_credit):
        my_x = lax.axis_index("x")
        my_y = lax.axis_index("y")
        my_z = lax.axis_index("z")
        yp = (my_x, 1 - my_y, my_z)
        xp = (1 - my_x, my_y, my_z)

        mine = my_x * HALF
        other = (1 - my_x) * HALF

        barrier_sem = pltpu.get_barrier_semaphore()
        for nbr in (yp, xp):
            pl.semaphore_signal(barrier_sem, inc=1, device_id=nbr,
                                device_id_type=MESH)
        pl.semaphore_wait(barrier_sem, 2)

        def mk_load(i):
            s = i % 2
            return pltpu.make_async_copy(
                x_ref.at[pl.ds(mine + i * CH, CH)], loc.at[s],
                load_sems.at[s])

        def mk_yrdma(i):
            s = i % 2
            return pltpu.make_async_remote_copy(
                src_ref=ysend.at[s], dst_ref=yrecv.at[s],
                send_sem=ysend_sems.at[s], recv_sem=yrecv_sems.at[s],
                device_id=yp, device_id_type=MESH)

        def mk_xrdma(i):
            s = i % 2
            return pltpu.make_async_remote_copy(
                src_ref=xsend.at[s], dst_ref=xrecv.at[s],
                send_sem=xsend_sems.at[s], recv_sem=xrecv_sems.at[s],
                device_id=xp, device_id_type=MESH)

        def mk_store_mine(i):
            s = i % 2
            return pltpu.make_async_copy(
                xsend.at[s], out_ref.at[pl.ds(mine + i * CH, CH)],
                smine_sems.at[s])

        def mk_store_other(i):
            s = i % 2
            return pltpu.make_async_copy(
                xrecv.at[s], out_ref.at[pl.ds(other + i * CH, CH)],
                sother_sems.at[s])

        def conv(i):
            s = i % 2
            if i >= 2:
                mk_yrdma(i - 2).wait_send()
            mk_load(i).wait()
            ysend[s, :, :] = loc[s, :, :].astype(jnp.bfloat16)

        def ystart(i):
            if i >= 2:
                pl.semaphore_wait(y_credit, 1)
            mk_yrdma(i).start()

        def reduce_(i):
            s = i % 2
            if i >= 2:
                mk_xrdma(i - 2).wait_send()
                mk_store_mine(i - 2).wait()
            mk_yrdma(i).wait_recv()
            xsend[s, :, :] = (loc[s, :, :] + yrecv[s, :, :].astype(jnp.float32)
                              ).astype(jnp.bfloat16)
            if i < NC - 2:
                pl.semaphore_signal(y_credit, inc=1, device_id=yp,
                                    device_id_type=MESH)

        def xstart(i):
            mk_store_mine(i).start()
            if i >= 2:
                pl.semaphore_wait(x_credit, 1)
            mk_xrdma(i).start()

        def finish(i):
            mk_xrdma(i).wait_recv()
            st = mk_store_other(i)
            st.start()
            st.wait()
            if i < NC - 2:
                pl.semaphore_signal(x_credit, inc=1, device_id=xp,
                                    device_id_type=MESH)

        mk_load(0).start()
        conv(0)
        mk_load(1).start()
        ystart(0)

        for i in range(NC):
            if i + 1 < NC:
                conv(i + 1)
                ystart(i + 1)
            reduce_(i)
            if i + 2 < NC:
                mk_load(i + 2).start()
            xstart(i)
            if i >= 1:
                finish(i - 1)
        finish(NC - 1)

        for i in (NC - 2, NC - 1):
            mk_yrdma(i).wait_send()
            mk_xrdma(i).wait_send()
            mk_store_mine(i).wait()

    return pl.pallas_call(
        body,
        out_shape=jax.ShapeDtypeStruct((M, N), jnp.bfloat16),
        in_specs=[pl.BlockSpec(memory_space=pl.ANY)],
        out_specs=pl.BlockSpec(memory_space=pl.ANY),
        scratch_shapes=[
            pltpu.VMEM((2, CH, N), jnp.float32),
            pltpu.VMEM((2, CH, N), jnp.bfloat16),
            pltpu.VMEM((2, CH, N), jnp.bfloat16),
            pltpu.VMEM((2, CH, N), jnp.bfloat16),
            pltpu.VMEM((2, CH, N), jnp.bfloat16),
            pltpu.SemaphoreType.DMA((2,)),
            pltpu.SemaphoreType.DMA((2,)),
            pltpu.SemaphoreType.DMA((2,)),
            pltpu.SemaphoreType.DMA((2,)),
            pltpu.SemaphoreType.DMA((2,)),
            pltpu.SemaphoreType.DMA((2,)),
            pltpu.SemaphoreType.DMA((2,)),
            pltpu.SemaphoreType.REGULAR,
            pltpu.SemaphoreType.REGULAR,
        ],
        compiler_params=pltpu.CompilerParams(
            collective_id=0, vmem_limit_bytes=100 * 1024 * 1024),
    )(x)


# device time: 435191 ns/iter; 1.0571x vs baseline; 1.0571x over previous
import jax
import jax.numpy as jnp
from jax import lax
from jax.experimental import pallas as pl
from jax.experimental.pallas import tpu as pltpu

M, N = 32768, 1024
HALF = M // 2
CH = 1024
NC = HALF // CH
NSLOT = 3

MESH = pl.DeviceIdType.MESH


def kernel(x):
    def body(x_ref, out_ref, loc, ysend, yrecv, xsend, xrecv,
             load_sems, smine_sems, sother_sems,
             ysend_sems, yrecv_sems, xsend_sems, xrecv_sems,
             y_credit, x_credit):
        my_x = lax.axis_index("x")
        my_y = lax.axis_index("y")
        my_z = lax.axis_index("z")
        yp = (my_x, 1 - my_y, my_z)
        xp = (1 - my_x, my_y, my_z)

        mine = my_x * HALF
        other = (1 - my_x) * HALF

        barrier_sem = pltpu.get_barrier_semaphore()
        for nbr in (yp, xp):
            pl.semaphore_signal(barrier_sem, inc=1, device_id=nbr,
                                device_id_type=MESH)
        pl.semaphore_wait(barrier_sem, 2)

        def mk_load(i):
            s = i % NSLOT
            return pltpu.make_async_copy(
                x_ref.at[pl.ds(mine + i * CH, CH)], loc.at[s],
                load_sems.at[s])

        def mk_yrdma(i):
            s = i % NSLOT
            return pltpu.make_async_remote_copy(
                src_ref=ysend.at[s], dst_ref=yrecv.at[s],
                send_sem=ysend_sems.at[s], recv_sem=yrecv_sems.at[s],
                device_id=yp, device_id_type=MESH)

        def mk_xrdma(i):
            s = i % NSLOT
            return pltpu.make_async_remote_copy(
                src_ref=xsend.at[s], dst_ref=xrecv.at[s],
                send_sem=xsend_sems.at[s], recv_sem=xrecv_sems.at[s],
                device_id=xp, device_id_type=MESH)

        def mk_store_mine(i):
            s = i % NSLOT
            return pltpu.make_async_copy(
                xsend.at[s], out_ref.at[pl.ds(mine + i * CH, CH)],
                smine_sems.at[s])

        def mk_store_other(i):
            s = i % NSLOT
            return pltpu.make_async_copy(
                xrecv.at[s], out_ref.at[pl.ds(other + i * CH, CH)],
                sother_sems.at[s])

        def conv(i):
            s = i % NSLOT
            if i >= NSLOT:
                mk_yrdma(i - NSLOT).wait_send()
            mk_load(i).wait()
            ysend[s, :, :] = loc[s, :, :].astype(jnp.bfloat16)

        def ystart(i):
            if i >= NSLOT:
                pl.semaphore_wait(y_credit, 1)
            mk_yrdma(i).start()

        def reduce_(i):
            s = i % NSLOT
            if i >= NSLOT:
                mk_xrdma(i - NSLOT).wait_send()
                mk_store_mine(i - NSLOT).wait()
            mk_yrdma(i).wait_recv()
            xsend[s, :, :] = (loc[s, :, :] + yrecv[s, :, :].astype(jnp.float32)
                              ).astype(jnp.bfloat16)
            if i < NC - NSLOT:
                pl.semaphore_signal(y_credit, inc=1, device_id=yp,
                                    device_id_type=MESH)

        def xstart(i):
            mk_store_mine(i).start()
            if i >= NSLOT:
                pl.semaphore_wait(x_credit, 1)
            mk_xrdma(i).start()

        def finish(i):
            mk_xrdma(i).wait_recv()
            st = mk_store_other(i)
            st.start()
            st.wait()
            if i < NC - NSLOT:
                pl.semaphore_signal(x_credit, inc=1, device_id=xp,
                                    device_id_type=MESH)

        mk_load(0).start()
        conv(0)
        for k in range(1, min(NSLOT, NC)):
            mk_load(k).start()
        ystart(0)

        for i in range(NC):
            if i + 1 < NC:
                conv(i + 1)
                ystart(i + 1)
            reduce_(i)
            if i + NSLOT < NC:
                mk_load(i + NSLOT).start()
            xstart(i)
            if i >= 1:
                finish(i - 1)
        finish(NC - 1)

        for i in range(max(NC - NSLOT, 0), NC):
            mk_yrdma(i).wait_send()
            mk_xrdma(i).wait_send()
            mk_store_mine(i).wait()

    return pl.pallas_call(
        body,
        out_shape=jax.ShapeDtypeStruct((M, N), jnp.bfloat16),
        in_specs=[pl.BlockSpec(memory_space=pl.ANY)],
        out_specs=pl.BlockSpec(memory_space=pl.ANY),
        scratch_shapes=[
            pltpu.VMEM((NSLOT, CH, N), jnp.float32),
            pltpu.VMEM((NSLOT, CH, N), jnp.bfloat16),
            pltpu.VMEM((NSLOT, CH, N), jnp.bfloat16),
            pltpu.VMEM((NSLOT, CH, N), jnp.bfloat16),
            pltpu.VMEM((NSLOT, CH, N), jnp.bfloat16),
            pltpu.SemaphoreType.DMA((NSLOT,)),
            pltpu.SemaphoreType.DMA((NSLOT,)),
            pltpu.SemaphoreType.DMA((NSLOT,)),
            pltpu.SemaphoreType.DMA((NSLOT,)),
            pltpu.SemaphoreType.DMA((NSLOT,)),
            pltpu.SemaphoreType.DMA((NSLOT,)),
            pltpu.SemaphoreType.DMA((NSLOT,)),
            pltpu.SemaphoreType.REGULAR,
            pltpu.SemaphoreType.REGULAR,
        ],
        compiler_params=pltpu.CompilerParams(
            collective_id=0, vmem_limit_bytes=100 * 1024 * 1024),
    )(x)


# device time: 434004 ns/iter; 1.0600x vs baseline; 1.0027x over previous
import jax
import jax.numpy as jnp
from jax import lax
from jax.experimental import pallas as pl
from jax.experimental.pallas import tpu as pltpu

M, N = 32768, 1024
HALF = M // 2
CH = 1024
NC = HALF // CH
NSLOT = 3

MESH = pl.DeviceIdType.MESH


def kernel(x):
    def body(x_ref, out_ref, loc, ysend, yrecv, xsend,
             load_sems, smine_sems,
             ysend_sems, yrecv_sems, xsend_sems, xrecv_sems,
             y_credit):
        my_x = lax.axis_index("x")
        my_y = lax.axis_index("y")
        my_z = lax.axis_index("z")
        yp = (my_x, 1 - my_y, my_z)
        xp = (1 - my_x, my_y, my_z)

        mine = my_x * HALF

        barrier_sem = pltpu.get_barrier_semaphore()
        for nbr in (yp, xp):
            pl.semaphore_signal(barrier_sem, inc=1, device_id=nbr,
                                device_id_type=MESH)
        pl.semaphore_wait(barrier_sem, 2)

        def mk_load(i):
            s = i % NSLOT
            return pltpu.make_async_copy(
                x_ref.at[pl.ds(mine + i * CH, CH)], loc.at[s],
                load_sems.at[s])

        def mk_yrdma(i):
            s = i % NSLOT
            return pltpu.make_async_remote_copy(
                src_ref=ysend.at[s], dst_ref=yrecv.at[s],
                send_sem=ysend_sems.at[s], recv_sem=yrecv_sems.at[s],
                device_id=yp, device_id_type=MESH)

        def mk_xrdma(i):
            s = i % NSLOT
            return pltpu.make_async_remote_copy(
                src_ref=xsend.at[s],
                dst_ref=out_ref.at[pl.ds(mine + i * CH, CH)],
                send_sem=xsend_sems.at[s], recv_sem=xrecv_sems.at[s],
                device_id=xp, device_id_type=MESH)

        def mk_store_mine(i):
            s = i % NSLOT
            return pltpu.make_async_copy(
                xsend.at[s], out_ref.at[pl.ds(mine + i * CH, CH)],
                smine_sems.at[s])

        def conv(i):
            s = i % NSLOT
            if i >= NSLOT:
                mk_yrdma(i - NSLOT).wait_send()
            mk_load(i).wait()
            ysend[s, :, :] = loc[s, :, :].astype(jnp.bfloat16)

        def ystart(i):
            if i >= NSLOT:
                pl.semaphore_wait(y_credit, 1)
            mk_yrdma(i).start()

        def reduce_(i):
            s = i % NSLOT
            if i >= NSLOT:
                mk_xrdma(i - NSLOT).wait_send()
                mk_store_mine(i - NSLOT).wait()
            mk_yrdma(i).wait_recv()
            xsend[s, :, :] = (loc[s, :, :] + yrecv[s, :, :].astype(jnp.float32)
                              ).astype(jnp.bfloat16)
            if i < NC - NSLOT:
                pl.semaphore_signal(y_credit, inc=1, device_id=yp,
                                    device_id_type=MESH)

        def xstart(i):
            mk_store_mine(i).start()
            mk_xrdma(i).start()

        def finish(i):
            mk_xrdma(i).wait_recv()

        mk_load(0).start()
        conv(0)
        for k in range(1, min(NSLOT, NC)):
            mk_load(k).start()
        ystart(0)

        for i in range(NC):
            if i + 1 < NC:
                conv(i + 1)
                ystart(i + 1)
            reduce_(i)
            if i + NSLOT < NC:
                mk_load(i + NSLOT).start()
            xstart(i)
            if i >= 1:
                finish(i - 1)
        finish(NC - 1)

        for i in range(max(NC - NSLOT, 0), NC):
            mk_yrdma(i).wait_send()
            mk_xrdma(i).wait_send()
            mk_store_mine(i).wait()

    return pl.pallas_call(
        body,
        out_shape=jax.ShapeDtypeStruct((M, N), jnp.bfloat16),
        in_specs=[pl.BlockSpec(memory_space=pl.ANY)],
        out_specs=pl.BlockSpec(memory_space=pl.ANY),
        scratch_shapes=[
            pltpu.VMEM((NSLOT, CH, N), jnp.float32),
            pltpu.VMEM((NSLOT, CH, N), jnp.bfloat16),
            pltpu.VMEM((NSLOT, CH, N), jnp.bfloat16),
            pltpu.VMEM((NSLOT, CH, N), jnp.bfloat16),
            pltpu.SemaphoreType.DMA((NSLOT,)),
            pltpu.SemaphoreType.DMA((NSLOT,)),
            pltpu.SemaphoreType.DMA((NSLOT,)),
            pltpu.SemaphoreType.DMA((NSLOT,)),
            pltpu.SemaphoreType.DMA((NSLOT,)),
            pltpu.SemaphoreType.DMA((NSLOT,)),
            pltpu.SemaphoreType.REGULAR,
        ],
        compiler_params=pltpu.CompilerParams(
            collective_id=0, vmem_limit_bytes=100 * 1024 * 1024),
    )(x)


# device time: 422292 ns/iter; 1.0894x vs baseline; 1.0277x over previous
import jax
import jax.numpy as jnp
from jax import lax
from jax.experimental import pallas as pl
from jax.experimental.pallas import tpu as pltpu

M, N = 32768, 1024
HALF = M // 2
CH = 512
NC = HALF // CH
NSLOT = 4

MESH = pl.DeviceIdType.MESH


def kernel(x):
    def body(x_ref, out_ref, loc, ysend, yrecv, xsend,
             load_sems, smine_sems,
             ysend_sems, yrecv_sems, xsend_sems, xrecv_sems,
             y_credit):
        my_x = lax.axis_index("x")
        my_y = lax.axis_index("y")
        my_z = lax.axis_index("z")
        yp = (my_x, 1 - my_y, my_z)
        xp = (1 - my_x, my_y, my_z)

        mine = my_x * HALF

        barrier_sem = pltpu.get_barrier_semaphore()
        for nbr in (yp, xp):
            pl.semaphore_signal(barrier_sem, inc=1, device_id=nbr,
                                device_id_type=MESH)
        pl.semaphore_wait(barrier_sem, 2)

        def mk_load(i):
            s = i % NSLOT
            return pltpu.make_async_copy(
                x_ref.at[pl.ds(mine + i * CH, CH)], loc.at[s],
                load_sems.at[s])

        def mk_yrdma(i):
            s = i % NSLOT
            return pltpu.make_async_remote_copy(
                src_ref=ysend.at[s], dst_ref=yrecv.at[s],
                send_sem=ysend_sems.at[s], recv_sem=yrecv_sems.at[s],
                device_id=yp, device_id_type=MESH)

        def mk_xrdma(i):
            s = i % NSLOT
            return pltpu.make_async_remote_copy(
                src_ref=xsend.at[s],
                dst_ref=out_ref.at[pl.ds(mine + i * CH, CH)],
                send_sem=xsend_sems.at[s], recv_sem=xrecv_sems.at[s],
                device_id=xp, device_id_type=MESH)

        def mk_store_mine(i):
            s = i % NSLOT
            return pltpu.make_async_copy(
                xsend.at[s], out_ref.at[pl.ds(mine + i * CH, CH)],
                smine_sems.at[s])

        def conv(i):
            s = i % NSLOT
            if i >= NSLOT:
                mk_yrdma(i - NSLOT).wait_send()
            mk_load(i).wait()
            ysend[s, :, :] = loc[s, :, :].astype(jnp.bfloat16)

        def ystart(i):
            if i >= NSLOT:
                pl.semaphore_wait(y_credit, 1)
            mk_yrdma(i).start()

        def reduce_(i):
            s = i % NSLOT
            if i >= NSLOT:
                mk_xrdma(i - NSLOT).wait_send()
                mk_store_mine(i - NSLOT).wait()
            mk_yrdma(i).wait_recv()
            xsend[s, :, :] = (loc[s, :, :] + yrecv[s, :, :].astype(jnp.float32)
                              ).astype(jnp.bfloat16)
            if i < NC - NSLOT:
                pl.semaphore_signal(y_credit, inc=1, device_id=yp,
                                    device_id_type=MESH)

        def xstart(i):
            mk_store_mine(i).start()
            mk_xrdma(i).start()

        def finish(i):
            mk_xrdma(i).wait_recv()

        mk_load(0).start()
        conv(0)
        for k in range(1, min(NSLOT, NC)):
            mk_load(k).start()
        ystart(0)

        for i in range(NC):
            if i + 1 < NC:
                conv(i + 1)
                ystart(i + 1)
            reduce_(i)
            if i + NSLOT < NC:
                mk_load(i + NSLOT).start()
            xstart(i)
            if i >= 1:
                finish(i - 1)
        finish(NC - 1)

        for i in range(max(NC - NSLOT, 0), NC):
            mk_yrdma(i).wait_send()
            mk_xrdma(i).wait_send()
            mk_store_mine(i).wait()

    return pl.pallas_call(
        body,
        out_shape=jax.ShapeDtypeStruct((M, N), jnp.bfloat16),
        in_specs=[pl.BlockSpec(memory_space=pl.ANY)],
        out_specs=pl.BlockSpec(memory_space=pl.ANY),
        scratch_shapes=[
            pltpu.VMEM((NSLOT, CH, N), jnp.float32),
            pltpu.VMEM((NSLOT, CH, N), jnp.bfloat16),
            pltpu.VMEM((NSLOT, CH, N), jnp.bfloat16),
            pltpu.VMEM((NSLOT, CH, N), jnp.bfloat16),
            pltpu.SemaphoreType.DMA((NSLOT,)),
            pltpu.SemaphoreType.DMA((NSLOT,)),
            pltpu.SemaphoreType.DMA((NSLOT,)),
            pltpu.SemaphoreType.DMA((NSLOT,)),
            pltpu.SemaphoreType.DMA((NSLOT,)),
            pltpu.SemaphoreType.DMA((NSLOT,)),
            pltpu.SemaphoreType.REGULAR,
        ],
        compiler_params=pltpu.CompilerParams(
            collective_id=0, vmem_limit_bytes=100 * 1024 * 1024),
    )(x)


# device time: 416676 ns/iter; 1.1040x vs baseline; 1.0135x over previous
import jax
import jax.numpy as jnp
from jax import lax
from jax.experimental import pallas as pl
from jax.experimental.pallas import tpu as pltpu

M, N = 32768, 1024
HALF = M // 2
CH = 256
NC = HALF // CH
NSLOT = 6

MESH = pl.DeviceIdType.MESH


def kernel(x):
    def body(x_ref, out_ref, loc, ysend, yrecv, xsend,
             load_sems, smine_sems,
             ysend_sems, yrecv_sems, xsend_sems, xrecv_sems,
             y_credit):
        my_x = lax.axis_index("x")
        my_y = lax.axis_index("y")
        my_z = lax.axis_index("z")
        yp = (my_x, 1 - my_y, my_z)
        xp = (1 - my_x, my_y, my_z)

        mine = my_x * HALF

        barrier_sem = pltpu.get_barrier_semaphore()
        for nbr in (yp, xp):
            pl.semaphore_signal(barrier_sem, inc=1, device_id=nbr,
                                device_id_type=MESH)
        pl.semaphore_wait(barrier_sem, 2)

        def mk_load(i):
            s = i % NSLOT
            return pltpu.make_async_copy(
                x_ref.at[pl.ds(mine + i * CH, CH)], loc.at[s],
                load_sems.at[s])

        def mk_yrdma(i):
            s = i % NSLOT
            return pltpu.make_async_remote_copy(
                src_ref=ysend.at[s], dst_ref=yrecv.at[s],
                send_sem=ysend_sems.at[s], recv_sem=yrecv_sems.at[s],
                device_id=yp, device_id_type=MESH)

        def mk_xrdma(i):
            s = i % NSLOT
            return pltpu.make_async_remote_copy(
                src_ref=xsend.at[s],
                dst_ref=out_ref.at[pl.ds(mine + i * CH, CH)],
                send_sem=xsend_sems.at[s], recv_sem=xrecv_sems.at[s],
                device_id=xp, device_id_type=MESH)

        def mk_store_mine(i):
            s = i % NSLOT
            return pltpu.make_async_copy(
                xsend.at[s], out_ref.at[pl.ds(mine + i * CH, CH)],
                smine_sems.at[s])

        def conv(i):
            s = i % NSLOT
            if i >= NSLOT:
                mk_yrdma(i - NSLOT).wait_send()
            mk_load(i).wait()
            ysend[s, :, :] = loc[s, :, :].astype(jnp.bfloat16)

        def ystart(i):
            if i >= NSLOT:
                pl.semaphore_wait(y_credit, 1)
            mk_yrdma(i).start()

        def reduce_(i):
            s = i % NSLOT
            if i >= NSLOT:
                mk_xrdma(i - NSLOT).wait_send()
                mk_store_mine(i - NSLOT).wait()
            mk_yrdma(i).wait_recv()
            xsend[s, :, :] = (loc[s, :, :] + yrecv[s, :, :].astype(jnp.float32)
                              ).astype(jnp.bfloat16)
            if i < NC - NSLOT:
                pl.semaphore_signal(y_credit, inc=1, device_id=yp,
                                    device_id_type=MESH)

        def xstart(i):
            mk_store_mine(i).start()
            mk_xrdma(i).start()

        def finish(i):
            mk_xrdma(i).wait_recv()

        mk_load(0).start()
        conv(0)
        for k in range(1, min(NSLOT, NC)):
            mk_load(k).start()
        ystart(0)

        for i in range(NC):
            if i + 1 < NC:
                conv(i + 1)
                ystart(i + 1)
            reduce_(i)
            if i + NSLOT < NC:
                mk_load(i + NSLOT).start()
            xstart(i)
            if i >= 1:
                finish(i - 1)
        finish(NC - 1)

        for i in range(max(NC - NSLOT, 0), NC):
            mk_yrdma(i).wait_send()
            mk_xrdma(i).wait_send()
            mk_store_mine(i).wait()

    return pl.pallas_call(
        body,
        out_shape=jax.ShapeDtypeStruct((M, N), jnp.bfloat16),
        in_specs=[pl.BlockSpec(memory_space=pl.ANY)],
        out_specs=pl.BlockSpec(memory_space=pl.ANY),
        scratch_shapes=[
            pltpu.VMEM((NSLOT, CH, N), jnp.float32),
            pltpu.VMEM((NSLOT, CH, N), jnp.bfloat16),
            pltpu.VMEM((NSLOT, CH, N), jnp.bfloat16),
            pltpu.VMEM((NSLOT, CH, N), jnp.bfloat16),
            pltpu.SemaphoreType.DMA((NSLOT,)),
            pltpu.SemaphoreType.DMA((NSLOT,)),
            pltpu.SemaphoreType.DMA((NSLOT,)),
            pltpu.SemaphoreType.DMA((NSLOT,)),
            pltpu.SemaphoreType.DMA((NSLOT,)),
            pltpu.SemaphoreType.DMA((NSLOT,)),
            pltpu.SemaphoreType.REGULAR,
        ],
        compiler_params=pltpu.CompilerParams(
            collective_id=0, vmem_limit_bytes=100 * 1024 * 1024),
    )(x)
